# Initial kernel scaffold; baseline (speedup 1.0000x reference)
import jax
import jax.numpy as jnp
from jax.experimental import pallas as pl


def kernel(x, edge_index, batch_idx, W1r, b1, W1s, W2r, b2, W2s, W3r, b3, W3s, Wf1, bf1, Wf2, bf2, mask1, mask2):
    raise NotImplementedError("write your pallas kernel here")


# speedup vs baseline: 2.2149x; 2.2149x over previous
import functools

import jax
import jax.numpy as jnp
from jax import lax
from jax.experimental import pallas as pl
from jax.experimental.pallas import tpu as pltpu
from jax.experimental.pallas import tpu_sc as plsc

_N = 10000
_E = 320000
_F = 128
_H = 66
_G = 64
_HP = 128
_CNT = 66
_NC, _NS = 1, 16
_NW = _NC * _NS
_NP = 10240
_CW = 128
_EP = 327680
_RPW = _EP // _CW // _NW
_RPS = 2
_STEPS = _RPW // _RPS
_RT = _NP // _NS
_ZR = _RT // 4

_f32 = jnp.float32


def _seg_body(y, src2, dst2, out, idx_s, idx_d, rows, accum, gsem):
    s = lax.axis_index("s")
    w = s

    def _zb(i, carry):
        rows[i // 8, pl.ds((i % 8) * 16, 16)] = jnp.zeros((16,), _f32)
        return carry
    lax.fori_loop(0, _ZR * 8, _zb, 0)
    base = s * _RT
    for t in range(_RT // _ZR):
        pltpu.sync_copy(rows.at[pl.ds(0, _ZR)], accum.at[pl.ds(base + t * _ZR, _ZR)])
    plsc.subcore_barrier()

    def _step(t, carry):
        row0 = w * _RPW + t * _RPS
        pltpu.sync_copy(src2.at[pl.ds(row0, _RPS)], idx_s)
        pltpu.sync_copy(dst2.at[pl.ds(row0, _RPS)], idx_d)
        cps = [
            pltpu.async_copy(y.at[idx_s.at[j]], rows.at[pl.ds(j * _CW, _CW)], gsem)
            for j in range(_RPS)
        ]
        for cp in cps:
            cp.wait()
        for j in range(_RPS):
            pltpu.sync_copy(rows.at[pl.ds(j * _CW, _CW)], accum.at[idx_d.at[j]],
                            add=True)
        return carry
    lax.fori_loop(0, _STEPS, _step, 0)
    plsc.subcore_barrier()

    for t in range(_RT // _ZR):
        r0 = s * _RT + t * _ZR
        pltpu.sync_copy(accum.at[pl.ds(r0, _ZR)], rows.at[pl.ds(0, _ZR)])
        pltpu.sync_copy(rows.at[pl.ds(0, _ZR)], out.at[pl.ds(r0, _ZR)])


_seg_sum = functools.partial(
    pl.kernel,
    out_type=jax.ShapeDtypeStruct((_NP, _HP), _f32),
    mesh=plsc.VectorSubcoreMesh(core_axis_name="c", subcore_axis_name="s",
                                num_cores=_NC, num_subcores=_NS),
    scratch_types=[
        pltpu.VMEM((_RPS, _CW), jnp.int32),
        pltpu.VMEM((_RPS, _CW), jnp.int32),
        pltpu.VMEM((_RPS * _CW, _HP), _f32),
        pltpu.VMEM_SHARED((_NP, _HP), _f32),
        pltpu.SemaphoreType.DMA,
    ],
)(_seg_body)


def _e66_row():
    return (lax.broadcasted_iota(jnp.int32, (1, _HP), 1) == _CNT).astype(_f32)


def _proj(h_ref, wr_ref, ws_ref, b_ref, y_ref, z_ref):
    h = h_ref[...]
    y_ref[...] = jnp.dot(h, wr_ref[...], preferred_element_type=_f32) + _e66_row()
    z_ref[...] = jnp.dot(h, ws_ref[...], preferred_element_type=_f32) + b_ref[...]


def _combine(p_ref, z_ref, h_ref):
    sseg = p_ref[...]
    e66c = (lax.broadcasted_iota(jnp.int32, (_HP, 1), 0) == _CNT).astype(_f32)
    cnt = jnp.dot(sseg, e66c, preferred_element_type=_f32)
    inv = 1.0 / jnp.maximum(cnt, 1.0)
    h_ref[...] = jnp.maximum(sseg * inv + z_ref[...], 0.0)


def _pool_head(h_ref, bi_ref, m1_ref, wf1_ref, bf1_ref, m2_ref,
               wf2_ref, bf2_ref, o_ref):
    h = h_ref[...]
    oh = (bi_ref[...] == lax.broadcasted_iota(jnp.int32, (_NP, _G), 1)).astype(_f32)
    dn = (((0,), (0,)), ((), ()))
    pool = lax.dot_general(oh, h, dn, preferred_element_type=_f32)
    cnt_g = lax.dot_general(oh, jnp.ones((_NP, 1), _f32), dn,
                            preferred_element_type=_f32)
    f = pool * (1.0 / jnp.maximum(cnt_g, 1.0))
    f = f * m1_ref[...] * 2.0
    f = jnp.dot(f, wf1_ref[...], preferred_element_type=_f32) + bf1_ref[...]
    f = jnp.where(f > 0, f, 0.01 * f)
    f = f * m2_ref[...] * 2.0
    f = jnp.dot(f, wf2_ref[...], preferred_element_type=_f32) + bf2_ref[...]
    o_ref[...] = jnp.where(f > 0, f, 0.01 * f)


def _call_proj(h, wr, ws, b):
    return pl.pallas_call(
        _proj,
        out_shape=[jax.ShapeDtypeStruct((_NP, _HP), _f32),
                   jax.ShapeDtypeStruct((_NP, _HP), _f32)],
    )(h, wr, ws, b)


def _call_combine(p, z):
    return pl.pallas_call(
        _combine,
        out_shape=jax.ShapeDtypeStruct((_NP, _HP), _f32),
    )(p, z)


def _call_pool_head(h, bi, m1, wf1, bf1, m2, wf2, bf2):
    return pl.pallas_call(
        _pool_head,
        out_shape=jax.ShapeDtypeStruct((_G, 1), _f32),
    )(h, bi, m1, wf1, bf1, m2, wf2, bf2)


def kernel(x, edge_index, batch_idx, W1r, b1, W1s, W2r, b2, W2s, W3r, b3, W3s,
           Wf1, bf1, Wf2, bf2, mask1, mask2):
    xp = jnp.pad(x, ((0, _NP - _N), (0, 0)))
    pad = jnp.full((_EP - _E,), _N, jnp.int32)
    src2 = jnp.concatenate([edge_index[0], pad]).reshape(_EP // _CW, _CW)
    dst2 = jnp.concatenate([edge_index[1], pad]).reshape(_EP // _CW, _CW)
    bi = jnp.pad(batch_idx, (0, _NP - _N), constant_values=-1).reshape(_NP, 1)

    def padw(w):
        return jnp.pad(w, ((0, _HP - w.shape[0] if w.shape[0] == _H else 0),
                           (0, _HP - w.shape[1])))
    w1r, w1s = padw(W1r), padw(W1s)
    w2r, w2s = padw(W2r), padw(W2s)
    w3r, w3s = padw(W3r), padw(W3s)
    b1p = jnp.pad(b1, (0, _HP - _H)).reshape(1, _HP)
    b2p = jnp.pad(b2, (0, _HP - _H)).reshape(1, _HP)
    b3p = jnp.pad(b3, (0, _HP - _H)).reshape(1, _HP)
    m1p = jnp.pad(mask1, ((0, 0), (0, _HP - _H)))
    wf1p = jnp.pad(Wf1, ((0, _HP - _H), (0, 0)))
    bf1p = bf1.reshape(1, 30)
    bf2p = bf2.reshape(1, 1)

    Wr = jnp.stack([w1r, w2r, w3r])
    Ws = jnp.stack([w1s, w2s, w3s])
    Bs = jnp.stack([b1p, b2p, b3p])

    def layer(l, h):
        y, z = _call_proj(h, Wr[l], Ws[l], Bs[l])
        p = _seg_sum(y, src2, dst2)
        return _call_combine(p, z)

    h = lax.fori_loop(0, 3, layer, xp)
    return _call_pool_head(h, bi, m1p, wf1p, bf1p, mask2, Wf2, bf2p)


# speedup vs baseline: 2.9517x; 1.3326x over previous
import functools

import jax
import jax.numpy as jnp
from jax import lax
from jax.experimental import pallas as pl
from jax.experimental.pallas import tpu as pltpu
from jax.experimental.pallas import tpu_sc as plsc

_N = 10000
_E = 320000
_F = 128
_H = 66
_G = 64
_HP = 128
_CNT = 66
_NC, _NS = 2, 16
_NW = _NC * _NS
_NP = 10240
_CW = 128
_EP = 327680
_RPW = _EP // _CW // _NW
_RPS = 2
_STEPS = _RPW // _RPS
_RT = _NP // _NS
_ZR = _RT // 4

_f32 = jnp.float32


def _seg_body(y, src2, dst2, out, idx_s, idx_d, rows, accum, gsem):
    c = lax.axis_index("c")
    s = lax.axis_index("s")
    w = s * _NC + c

    def _zb(i, carry):
        rows[i // 8, pl.ds((i % 8) * 16, 16)] = jnp.zeros((16,), _f32)
        return carry
    lax.fori_loop(0, _ZR * 8, _zb, 0)
    base = s * _RT
    for t in range(_RT // _ZR):
        pltpu.sync_copy(rows.at[pl.ds(0, _ZR)], accum.at[pl.ds(base + t * _ZR, _ZR)])
    plsc.subcore_barrier()

    def _step(t, carry):
        row0 = w * _RPW + t * _RPS
        pltpu.sync_copy(src2.at[pl.ds(row0, _RPS)], idx_s)
        pltpu.sync_copy(dst2.at[pl.ds(row0, _RPS)], idx_d)
        cps = [
            pltpu.async_copy(y.at[idx_s.at[j]], rows.at[pl.ds(j * _CW, _CW)], gsem)
            for j in range(_RPS)
        ]
        for cp in cps:
            cp.wait()
        for j in range(_RPS):
            pltpu.sync_copy(rows.at[pl.ds(j * _CW, _CW)], accum.at[idx_d.at[j]],
                            add=True)
        return carry
    lax.fori_loop(0, _STEPS, _step, 0)
    plsc.subcore_barrier()

    for t in range(_RT // _ZR):
        r0 = s * _RT + t * _ZR
        pltpu.sync_copy(accum.at[pl.ds(r0, _ZR)], rows.at[pl.ds(0, _ZR)])
        pltpu.sync_copy(rows.at[pl.ds(0, _ZR)], out.at[c, pl.ds(r0, _ZR)])


_seg_sum = functools.partial(
    pl.kernel,
    out_type=jax.ShapeDtypeStruct((_NC, _NP, _HP), _f32),
    mesh=plsc.VectorSubcoreMesh(core_axis_name="c", subcore_axis_name="s",
                                num_cores=_NC, num_subcores=_NS),
    scratch_types=[
        pltpu.VMEM((_RPS, _CW), jnp.int32),
        pltpu.VMEM((_RPS, _CW), jnp.int32),
        pltpu.VMEM((_RPS * _CW, _HP), _f32),
        pltpu.VMEM_SHARED((_NP, _HP), _f32),
        pltpu.SemaphoreType.DMA,
    ],
)(_seg_body)


def _e66_row():
    return (lax.broadcasted_iota(jnp.int32, (1, _HP), 1) == _CNT).astype(_f32)


def _proj(h_ref, wr_ref, ws_ref, b_ref, y_ref, z_ref):
    h = h_ref[...]
    y_ref[...] = jnp.dot(h, wr_ref[...], preferred_element_type=_f32) + _e66_row()
    z_ref[...] = jnp.dot(h, ws_ref[...], preferred_element_type=_f32) + b_ref[...]


def _combine(p_ref, z_ref, h_ref):
    sseg = p_ref[0] + p_ref[1]
    e66c = (lax.broadcasted_iota(jnp.int32, (_HP, 1), 0) == _CNT).astype(_f32)
    cnt = jnp.dot(sseg, e66c, preferred_element_type=_f32)
    inv = 1.0 / jnp.maximum(cnt, 1.0)
    h_ref[...] = jnp.maximum(sseg * inv + z_ref[...], 0.0)


def _pool_head(h_ref, bi_ref, m1_ref, wf1_ref, bf1_ref, m2_ref,
               wf2_ref, bf2_ref, o_ref):
    h = h_ref[...]
    oh = (bi_ref[...] == lax.broadcasted_iota(jnp.int32, (_NP, _G), 1)).astype(_f32)
    dn = (((0,), (0,)), ((), ()))
    pool = lax.dot_general(oh, h, dn, preferred_element_type=_f32)
    cnt_g = lax.dot_general(oh, jnp.ones((_NP, 1), _f32), dn,
                            preferred_element_type=_f32)
    f = pool * (1.0 / jnp.maximum(cnt_g, 1.0))
    f = f * m1_ref[...] * 2.0
    f = jnp.dot(f, wf1_ref[...], preferred_element_type=_f32) + bf1_ref[...]
    f = jnp.where(f > 0, f, 0.01 * f)
    f = f * m2_ref[...] * 2.0
    f = jnp.dot(f, wf2_ref[...], preferred_element_type=_f32) + bf2_ref[...]
    o_ref[...] = jnp.where(f > 0, f, 0.01 * f)


def _call_proj(h, wr, ws, b):
    return pl.pallas_call(
        _proj,
        out_shape=[jax.ShapeDtypeStruct((_NP, _HP), _f32),
                   jax.ShapeDtypeStruct((_NP, _HP), _f32)],
    )(h, wr, ws, b)


def _call_combine(p, z):
    return pl.pallas_call(
        _combine,
        out_shape=jax.ShapeDtypeStruct((_NP, _HP), _f32),
    )(p, z)


def _call_pool_head(h, bi, m1, wf1, bf1, m2, wf2, bf2):
    return pl.pallas_call(
        _pool_head,
        out_shape=jax.ShapeDtypeStruct((_G, 1), _f32),
    )(h, bi, m1, wf1, bf1, m2, wf2, bf2)


def kernel(x, edge_index, batch_idx, W1r, b1, W1s, W2r, b2, W2s, W3r, b3, W3s,
           Wf1, bf1, Wf2, bf2, mask1, mask2):
    xp = jnp.pad(x, ((0, _NP - _N), (0, 0)))
    pad = jnp.full((_EP - _E,), _N, jnp.int32)
    src2 = jnp.concatenate([edge_index[0], pad]).reshape(_EP // _CW, _CW)
    dst2 = jnp.concatenate([edge_index[1], pad]).reshape(_EP // _CW, _CW)
    bi = jnp.pad(batch_idx, (0, _NP - _N), constant_values=-1).reshape(_NP, 1)

    def padw(w):
        return jnp.pad(w, ((0, _HP - w.shape[0] if w.shape[0] == _H else 0),
                           (0, _HP - w.shape[1])))
    w1r, w1s = padw(W1r), padw(W1s)
    w2r, w2s = padw(W2r), padw(W2s)
    w3r, w3s = padw(W3r), padw(W3s)
    b1p = jnp.pad(b1, (0, _HP - _H)).reshape(1, _HP)
    b2p = jnp.pad(b2, (0, _HP - _H)).reshape(1, _HP)
    b3p = jnp.pad(b3, (0, _HP - _H)).reshape(1, _HP)
    m1p = jnp.pad(mask1, ((0, 0), (0, _HP - _H)))
    wf1p = jnp.pad(Wf1, ((0, _HP - _H), (0, 0)))
    bf1p = bf1.reshape(1, 30)
    bf2p = bf2.reshape(1, 1)

    Wr = jnp.stack([w1r, w2r, w3r])
    Ws = jnp.stack([w1s, w2s, w3s])
    Bs = jnp.stack([b1p, b2p, b3p])

    def layer(l, h):
        y, z = _call_proj(h, Wr[l], Ws[l], Bs[l])
        p = _seg_sum(y, src2, dst2)
        return _call_combine(p, z)

    h = lax.fori_loop(0, 3, layer, xp)
    return _call_pool_head(h, bi, m1p, wf1p, bf1p, mask2, Wf2, bf2p)


# speedup vs baseline: 3.3612x; 1.1387x over previous
import functools

import jax
import jax.numpy as jnp
from jax import lax
from jax.experimental import pallas as pl
from jax.experimental.pallas import tpu as pltpu
from jax.experimental.pallas import tpu_sc as plsc

_N = 10000
_E = 320000
_F = 128
_H = 66
_G = 64
_HP = 128
_CNT = 66
_NC, _NS = 2, 16
_NW = _NC * _NS
_NP = 10240
_CW = 128
_EP = 327680
_RPW = _EP // _CW // _NW
_RPS = 2
_STEPS = _RPW // _RPS
_RT = _NP // _NS
_ZR = _RT // 4

_f32 = jnp.float32


def _seg_body(y, src2, dst2, out, idx_s0, idx_d0, idx_s1, idx_d1,
              rows0, rows1, accum, sem0, sem1):
    c = lax.axis_index("c")
    s = lax.axis_index("s")
    w = s * _NC + c

    def _zb(i, carry):
        rows0[i // 8, pl.ds((i % 8) * 16, 16)] = jnp.zeros((16,), _f32)
        return carry
    lax.fori_loop(0, _CW * 8, _zb, 0)
    base = s * _RT
    for t in range(_RT // _CW):
        pltpu.sync_copy(rows0, accum.at[pl.ds(base + t * _CW, _CW)])
    plsc.subcore_barrier()

    wbase = w * _RPW
    pltpu.sync_copy(src2.at[pl.ds(wbase, 1)], idx_s0)
    pltpu.sync_copy(dst2.at[pl.ds(wbase, 1)], idx_d0)
    pltpu.async_copy(y.at[idx_s0.at[0]], rows0, sem0)

    def _step(g, carry):
        c0 = 2 * g
        pltpu.sync_copy(src2.at[pl.ds(wbase + c0 + 1, 1)], idx_s1)
        pltpu.sync_copy(dst2.at[pl.ds(wbase + c0 + 1, 1)], idx_d1)
        pltpu.async_copy(y.at[idx_s1.at[0]], rows1, sem1)
        pltpu.make_async_copy(y.at[idx_s0.at[0]], rows0, sem0).wait()
        pltpu.sync_copy(rows0, accum.at[idx_d0.at[0]], add=True)

        @pl.when(c0 + 2 < _RPW)
        def _():
            pltpu.sync_copy(src2.at[pl.ds(wbase + c0 + 2, 1)], idx_s0)
            pltpu.sync_copy(dst2.at[pl.ds(wbase + c0 + 2, 1)], idx_d0)
            pltpu.async_copy(y.at[idx_s0.at[0]], rows0, sem0)
        pltpu.make_async_copy(y.at[idx_s1.at[0]], rows1, sem1).wait()
        pltpu.sync_copy(rows1, accum.at[idx_d1.at[0]], add=True)
        return carry
    lax.fori_loop(0, _RPW // 2, _step, 0)
    plsc.subcore_barrier()

    for t in range(_RT // _CW):
        r0 = s * _RT + t * _CW
        pltpu.sync_copy(accum.at[pl.ds(r0, _CW)], rows0)
        pltpu.sync_copy(rows0, out.at[c, pl.ds(r0, _CW)])


_seg_sum = functools.partial(
    pl.kernel,
    out_type=jax.ShapeDtypeStruct((_NC, _NP, _HP), _f32),
    mesh=plsc.VectorSubcoreMesh(core_axis_name="c", subcore_axis_name="s",
                                num_cores=_NC, num_subcores=_NS),
    scratch_types=[
        pltpu.VMEM((1, _CW), jnp.int32),
        pltpu.VMEM((1, _CW), jnp.int32),
        pltpu.VMEM((1, _CW), jnp.int32),
        pltpu.VMEM((1, _CW), jnp.int32),
        pltpu.VMEM((_CW, _HP), _f32),
        pltpu.VMEM((_CW, _HP), _f32),
        pltpu.VMEM_SHARED((_NP, _HP), _f32),
        pltpu.SemaphoreType.DMA,
        pltpu.SemaphoreType.DMA,
    ],
)(_seg_body)


def _e66_row():
    return (lax.broadcasted_iota(jnp.int32, (1, _HP), 1) == _CNT).astype(_f32)


def _proj(h_ref, wr_ref, ws_ref, b_ref, y_ref, z_ref):
    h = h_ref[...]
    y_ref[...] = jnp.dot(h, wr_ref[...], preferred_element_type=_f32) + _e66_row()
    z_ref[...] = jnp.dot(h, ws_ref[...], preferred_element_type=_f32) + b_ref[...]


def _combine(p_ref, z_ref, h_ref):
    sseg = p_ref[0] + p_ref[1]
    e66c = (lax.broadcasted_iota(jnp.int32, (_HP, 1), 0) == _CNT).astype(_f32)
    cnt = jnp.dot(sseg, e66c, preferred_element_type=_f32)
    inv = 1.0 / jnp.maximum(cnt, 1.0)
    h_ref[...] = jnp.maximum(sseg * inv + z_ref[...], 0.0)


def _pool_head(h_ref, bi_ref, m1_ref, wf1_ref, bf1_ref, m2_ref,
               wf2_ref, bf2_ref, o_ref):
    h = h_ref[...]
    oh = (bi_ref[...] == lax.broadcasted_iota(jnp.int32, (_NP, _G), 1)).astype(_f32)
    dn = (((0,), (0,)), ((), ()))
    pool = lax.dot_general(oh, h, dn, preferred_element_type=_f32)
    cnt_g = lax.dot_general(oh, jnp.ones((_NP, 1), _f32), dn,
                            preferred_element_type=_f32)
    f = pool * (1.0 / jnp.maximum(cnt_g, 1.0))
    f = f * m1_ref[...] * 2.0
    f = jnp.dot(f, wf1_ref[...], preferred_element_type=_f32) + bf1_ref[...]
    f = jnp.where(f > 0, f, 0.01 * f)
    f = f * m2_ref[...] * 2.0
    f = jnp.dot(f, wf2_ref[...], preferred_element_type=_f32) + bf2_ref[...]
    o_ref[...] = jnp.where(f > 0, f, 0.01 * f)


def _call_proj(h, wr, ws, b):
    return pl.pallas_call(
        _proj,
        out_shape=[jax.ShapeDtypeStruct((_NP, _HP), _f32),
                   jax.ShapeDtypeStruct((_NP, _HP), _f32)],
    )(h, wr, ws, b)


def _call_combine(p, z):
    return pl.pallas_call(
        _combine,
        out_shape=jax.ShapeDtypeStruct((_NP, _HP), _f32),
    )(p, z)


def _call_pool_head(h, bi, m1, wf1, bf1, m2, wf2, bf2):
    return pl.pallas_call(
        _pool_head,
        out_shape=jax.ShapeDtypeStruct((_G, 1), _f32),
    )(h, bi, m1, wf1, bf1, m2, wf2, bf2)


def kernel(x, edge_index, batch_idx, W1r, b1, W1s, W2r, b2, W2s, W3r, b3, W3s,
           Wf1, bf1, Wf2, bf2, mask1, mask2):
    xp = jnp.pad(x, ((0, _NP - _N), (0, 0)))
    pad = jnp.full((_EP - _E,), _N, jnp.int32)
    src2 = jnp.concatenate([edge_index[0], pad]).reshape(_EP // _CW, _CW)
    dst2 = jnp.concatenate([edge_index[1], pad]).reshape(_EP // _CW, _CW)
    bi = jnp.pad(batch_idx, (0, _NP - _N), constant_values=-1).reshape(_NP, 1)

    def padw(w):
        return jnp.pad(w, ((0, _HP - w.shape[0] if w.shape[0] == _H else 0),
                           (0, _HP - w.shape[1])))
    w1r, w1s = padw(W1r), padw(W1s)
    w2r, w2s = padw(W2r), padw(W2s)
    w3r, w3s = padw(W3r), padw(W3s)
    b1p = jnp.pad(b1, (0, _HP - _H)).reshape(1, _HP)
    b2p = jnp.pad(b2, (0, _HP - _H)).reshape(1, _HP)
    b3p = jnp.pad(b3, (0, _HP - _H)).reshape(1, _HP)
    m1p = jnp.pad(mask1, ((0, 0), (0, _HP - _H)))
    wf1p = jnp.pad(Wf1, ((0, _HP - _H), (0, 0)))
    bf1p = bf1.reshape(1, 30)
    bf2p = bf2.reshape(1, 1)

    Wr = jnp.stack([w1r, w2r, w3r])
    Ws = jnp.stack([w1s, w2s, w3s])
    Bs = jnp.stack([b1p, b2p, b3p])

    def layer(l, h):
        y, z = _call_proj(h, Wr[l], Ws[l], Bs[l])
        p = _seg_sum(y, src2, dst2)
        return _call_combine(p, z)

    h = lax.fori_loop(0, 3, layer, xp)
    return _call_pool_head(h, bi, m1p, wf1p, bf1p, mask2, Wf2, bf2p)


# speedup vs baseline: 4.0277x; 1.1983x over previous
import functools

import jax
import jax.numpy as jnp
from jax import lax
from jax.experimental import pallas as pl
from jax.experimental.pallas import tpu as pltpu
from jax.experimental.pallas import tpu_sc as plsc

_N = 10000
_E = 320000
_F = 128
_H = 66
_G = 64
_HP = 128
_CNT = 66
_NC, _NS = 2, 16
_NW = _NC * _NS
_NP = 10240
_CW = 128
_EP = 327680
_RPW = _EP // _CW // _NW
_RPS = 2
_STEPS = _RPW // _RPS
_RT = _NP // _NS
_K0 = 120
_ZR = _RT // 4

_f32 = jnp.float32


def _seg_body(y, src2, dst2, out, idx_s0, idx_d0, idx_s1, idx_d1,
              rows0, rows1, accum, sem0, sem1):
    c = lax.axis_index("c")
    s = lax.axis_index("s")
    w = s * _NC + c

    def _zb(i, carry):
        rows0[i // 8, pl.ds((i % 8) * 16, 16)] = jnp.zeros((16,), _f32)
        return carry
    lax.fori_loop(0, _CW * 8, _zb, 0)
    base = s * _RT
    for t in range(_RT // _CW):
        pltpu.sync_copy(rows0, accum.at[pl.ds(base + t * _CW, _CW)])
    plsc.subcore_barrier()

    wbase = s * (2 * _RPW) + c * _K0
    nchunks = _K0 - (2 * _K0 - 2 * _RPW) * c
    pltpu.sync_copy(src2.at[pl.ds(wbase, 1)], idx_s0)
    pltpu.sync_copy(dst2.at[pl.ds(wbase, 1)], idx_d0)
    pltpu.async_copy(y.at[idx_s0.at[0]], rows0, sem0)

    def _step(g, carry):
        c0 = 2 * g
        pltpu.sync_copy(src2.at[pl.ds(wbase + c0 + 1, 1)], idx_s1)
        pltpu.sync_copy(dst2.at[pl.ds(wbase + c0 + 1, 1)], idx_d1)
        pltpu.async_copy(y.at[idx_s1.at[0]], rows1, sem1)
        pltpu.make_async_copy(y.at[idx_s0.at[0]], rows0, sem0).wait()
        pltpu.sync_copy(rows0, accum.at[idx_d0.at[0]], add=True)

        @pl.when(c0 + 2 < nchunks)
        def _():
            pltpu.sync_copy(src2.at[pl.ds(wbase + c0 + 2, 1)], idx_s0)
            pltpu.sync_copy(dst2.at[pl.ds(wbase + c0 + 2, 1)], idx_d0)
            pltpu.async_copy(y.at[idx_s0.at[0]], rows0, sem0)
        pltpu.make_async_copy(y.at[idx_s1.at[0]], rows1, sem1).wait()
        pltpu.sync_copy(rows1, accum.at[idx_d1.at[0]], add=True)
        return carry
    lax.fori_loop(0, nchunks // 2, _step, 0)
    plsc.subcore_barrier()

    for t in range(_RT // _CW):
        r0 = s * _RT + t * _CW
        pltpu.sync_copy(accum.at[pl.ds(r0, _CW)], rows0)
        pltpu.sync_copy(rows0, out.at[c, pl.ds(r0, _CW)])


_seg_sum = functools.partial(
    pl.kernel,
    out_type=jax.ShapeDtypeStruct((_NC, _NP, _HP), _f32),
    mesh=plsc.VectorSubcoreMesh(core_axis_name="c", subcore_axis_name="s",
                                num_cores=_NC, num_subcores=_NS),
    scratch_types=[
        pltpu.VMEM((1, _CW), jnp.int32),
        pltpu.VMEM((1, _CW), jnp.int32),
        pltpu.VMEM((1, _CW), jnp.int32),
        pltpu.VMEM((1, _CW), jnp.int32),
        pltpu.VMEM((_CW, _HP), _f32),
        pltpu.VMEM((_CW, _HP), _f32),
        pltpu.VMEM_SHARED((_NP, _HP), _f32),
        pltpu.SemaphoreType.DMA,
        pltpu.SemaphoreType.DMA,
    ],
)(_seg_body)


def _e66_row():
    return (lax.broadcasted_iota(jnp.int32, (1, _HP), 1) == _CNT).astype(_f32)


def _proj(h_ref, wr_ref, ws_ref, b_ref, y_ref, z_ref):
    h = h_ref[...]
    y_ref[...] = jnp.dot(h, wr_ref[...], preferred_element_type=_f32) + _e66_row()
    z_ref[...] = jnp.dot(h, ws_ref[...], preferred_element_type=_f32) + b_ref[...]


def _combine(p_ref, z_ref, h_ref):
    sseg = p_ref[0] + p_ref[1]
    e66c = (lax.broadcasted_iota(jnp.int32, (_HP, 1), 0) == _CNT).astype(_f32)
    cnt = jnp.dot(sseg, e66c, preferred_element_type=_f32)
    inv = 1.0 / jnp.maximum(cnt, 1.0)
    h_ref[...] = jnp.maximum(sseg * inv + z_ref[...], 0.0)


def _pool_head(h_ref, bi_ref, m1_ref, wf1_ref, bf1_ref, m2_ref,
               wf2_ref, bf2_ref, o_ref):
    h = h_ref[...]
    oh = (bi_ref[...] == lax.broadcasted_iota(jnp.int32, (_NP, _G), 1)).astype(_f32)
    dn = (((0,), (0,)), ((), ()))
    pool = lax.dot_general(oh, h, dn, preferred_element_type=_f32)
    cnt_g = lax.dot_general(oh, jnp.ones((_NP, 1), _f32), dn,
                            preferred_element_type=_f32)
    f = pool * (1.0 / jnp.maximum(cnt_g, 1.0))
    f = f * m1_ref[...] * 2.0
    f = jnp.dot(f, wf1_ref[...], preferred_element_type=_f32) + bf1_ref[...]
    f = jnp.where(f > 0, f, 0.01 * f)
    f = f * m2_ref[...] * 2.0
    f = jnp.dot(f, wf2_ref[...], preferred_element_type=_f32) + bf2_ref[...]
    o_ref[...] = jnp.where(f > 0, f, 0.01 * f)


def _call_proj(h, wr, ws, b):
    return pl.pallas_call(
        _proj,
        out_shape=[jax.ShapeDtypeStruct((_NP, _HP), _f32),
                   jax.ShapeDtypeStruct((_NP, _HP), _f32)],
    )(h, wr, ws, b)


def _call_combine(p, z):
    return pl.pallas_call(
        _combine,
        out_shape=jax.ShapeDtypeStruct((_NP, _HP), _f32),
    )(p, z)


def _call_pool_head(h, bi, m1, wf1, bf1, m2, wf2, bf2):
    return pl.pallas_call(
        _pool_head,
        out_shape=jax.ShapeDtypeStruct((_G, 1), _f32),
    )(h, bi, m1, wf1, bf1, m2, wf2, bf2)


def kernel(x, edge_index, batch_idx, W1r, b1, W1s, W2r, b2, W2s, W3r, b3, W3s,
           Wf1, bf1, Wf2, bf2, mask1, mask2):
    xp = jnp.pad(x, ((0, _NP - _N), (0, 0)))
    pad = jnp.full((_EP - _E,), _N, jnp.int32)
    src2 = jnp.concatenate([edge_index[0], pad]).reshape(_EP // _CW, _CW)
    dst2 = jnp.concatenate([edge_index[1], pad]).reshape(_EP // _CW, _CW)
    bi = jnp.pad(batch_idx, (0, _NP - _N), constant_values=-1).reshape(_NP, 1)

    def padw(w):
        return jnp.pad(w, ((0, _HP - w.shape[0] if w.shape[0] == _H else 0),
                           (0, _HP - w.shape[1])))
    w1r, w1s = padw(W1r), padw(W1s)
    w2r, w2s = padw(W2r), padw(W2s)
    w3r, w3s = padw(W3r), padw(W3s)
    b1p = jnp.pad(b1, (0, _HP - _H)).reshape(1, _HP)
    b2p = jnp.pad(b2, (0, _HP - _H)).reshape(1, _HP)
    b3p = jnp.pad(b3, (0, _HP - _H)).reshape(1, _HP)
    m1p = jnp.pad(mask1, ((0, 0), (0, _HP - _H)))
    wf1p = jnp.pad(Wf1, ((0, _HP - _H), (0, 0)))
    bf1p = bf1.reshape(1, 30)
    bf2p = bf2.reshape(1, 1)

    Wr = jnp.stack([w1r, w2r, w3r])
    Ws = jnp.stack([w1s, w2s, w3s])
    Bs = jnp.stack([b1p, b2p, b3p])

    def layer(l, h):
        y, z = _call_proj(h, Wr[l], Ws[l], Bs[l])
        p = _seg_sum(y, src2, dst2)
        return _call_combine(p, z)

    h = lax.fori_loop(0, 3, layer, xp)
    return _call_pool_head(h, bi, m1p, wf1p, bf1p, mask2, Wf2, bf2p)


# speedup vs baseline: 4.0633x; 1.0088x over previous
import functools

import jax
import jax.numpy as jnp
from jax import lax
from jax.experimental import pallas as pl
from jax.experimental.pallas import tpu as pltpu
from jax.experimental.pallas import tpu_sc as plsc

_N = 10000
_E = 320000
_F = 128
_H = 66
_G = 64
_HP = 128
_CNT = 66
_NC, _NS = 2, 16
_NW = _NC * _NS
_NP = 10240
_CW = 128
_EP = 327680
_RPW = _EP // _CW // _NW
_RPS = 2
_STEPS = _RPW // _RPS
_RT = _NP // _NS
_K0 = 120
_ZR = _RT // 4

_f32 = jnp.float32


def _seg_body(y, src2, dst2, out, idx_s, idx_d, rows0, rows1, accum,
              isem, sem0, sem1):
    c = lax.axis_index("c")
    s = lax.axis_index("s")

    def _zb(i, carry):
        rows0[i // 8, pl.ds((i % 8) * 16, 16)] = jnp.zeros((16,), _f32)
        return carry
    lax.fori_loop(0, _CW * 8, _zb, 0)
    base = s * _RT
    for t in range(_RT // _CW):
        pltpu.sync_copy(rows0, accum.at[pl.ds(base + t * _CW, _CW)])
    plsc.subcore_barrier()

    wbase = s * (2 * _RPW) + c * _K0
    nchunks = _K0 - (2 * _K0 - 2 * _RPW) * c

    def _grp(g, carry):
        r0 = wbase + g * 8
        pltpu.async_copy(src2.at[pl.ds(r0, 8)], idx_s, isem)
        pltpu.async_copy(dst2.at[pl.ds(r0, 8)], idx_d, isem)
        pltpu.make_async_copy(src2.at[pl.ds(r0, 8)], idx_s, isem).wait()
        pltpu.make_async_copy(dst2.at[pl.ds(r0, 8)], idx_d, isem).wait()
        pltpu.async_copy(y.at[idx_s.at[0]], rows0, sem0)
        for j in range(8):
            b_rows, b_sem = (rows0, sem0) if j % 2 == 0 else (rows1, sem1)
            n_rows, n_sem = (rows1, sem1) if j % 2 == 0 else (rows0, sem0)
            if j < 7:
                pltpu.async_copy(y.at[idx_s.at[j + 1]], n_rows, n_sem)
            pltpu.make_async_copy(y.at[idx_s.at[j]], b_rows, b_sem).wait()
            pltpu.sync_copy(b_rows, accum.at[idx_d.at[j]], add=True)
        return carry
    lax.fori_loop(0, nchunks // 8, _grp, 0)
    plsc.subcore_barrier()

    for t in range(_RT // _CW):
        r0 = s * _RT + t * _CW
        pltpu.sync_copy(accum.at[pl.ds(r0, _CW)], out.at[c, pl.ds(r0, _CW)])


_seg_sum = functools.partial(
    pl.kernel,
    out_type=jax.ShapeDtypeStruct((_NC, _NP, _HP), _f32),
    mesh=plsc.VectorSubcoreMesh(core_axis_name="c", subcore_axis_name="s",
                                num_cores=_NC, num_subcores=_NS),
    scratch_types=[
        pltpu.VMEM((8, _CW), jnp.int32),
        pltpu.VMEM((8, _CW), jnp.int32),
        pltpu.VMEM((_CW, _HP), _f32),
        pltpu.VMEM((_CW, _HP), _f32),
        pltpu.VMEM_SHARED((_NP, _HP), _f32),
        pltpu.SemaphoreType.DMA,
        pltpu.SemaphoreType.DMA,
        pltpu.SemaphoreType.DMA,
    ],
)(_seg_body)


def _e66_row():
    return (lax.broadcasted_iota(jnp.int32, (1, _HP), 1) == _CNT).astype(_f32)


def _proj(h_ref, wr_ref, ws_ref, b_ref, y_ref, z_ref):
    h = h_ref[...]
    y_ref[...] = jnp.dot(h, wr_ref[...], preferred_element_type=_f32) + _e66_row()
    z_ref[...] = jnp.dot(h, ws_ref[...], preferred_element_type=_f32) + b_ref[...]


def _combine(p_ref, z_ref, h_ref):
    sseg = p_ref[0] + p_ref[1]
    e66c = (lax.broadcasted_iota(jnp.int32, (_HP, 1), 0) == _CNT).astype(_f32)
    cnt = jnp.dot(sseg, e66c, preferred_element_type=_f32)
    inv = 1.0 / jnp.maximum(cnt, 1.0)
    h_ref[...] = jnp.maximum(sseg * inv + z_ref[...], 0.0)


def _pool_head(h_ref, bi_ref, m1_ref, wf1_ref, bf1_ref, m2_ref,
               wf2_ref, bf2_ref, o_ref):
    h = h_ref[...]
    oh = (bi_ref[...] == lax.broadcasted_iota(jnp.int32, (_NP, _G), 1)).astype(_f32)
    dn = (((0,), (0,)), ((), ()))
    pool = lax.dot_general(oh, h, dn, preferred_element_type=_f32)
    cnt_g = lax.dot_general(oh, jnp.ones((_NP, 1), _f32), dn,
                            preferred_element_type=_f32)
    f = pool * (1.0 / jnp.maximum(cnt_g, 1.0))
    f = f * m1_ref[...] * 2.0
    f = jnp.dot(f, wf1_ref[...], preferred_element_type=_f32) + bf1_ref[...]
    f = jnp.where(f > 0, f, 0.01 * f)
    f = f * m2_ref[...] * 2.0
    f = jnp.dot(f, wf2_ref[...], preferred_element_type=_f32) + bf2_ref[...]
    o_ref[...] = jnp.where(f > 0, f, 0.01 * f)


def _call_proj(h, wr, ws, b):
    return pl.pallas_call(
        _proj,
        out_shape=[jax.ShapeDtypeStruct((_NP, _HP), _f32),
                   jax.ShapeDtypeStruct((_NP, _HP), _f32)],
    )(h, wr, ws, b)


def _call_combine(p, z):
    return pl.pallas_call(
        _combine,
        out_shape=jax.ShapeDtypeStruct((_NP, _HP), _f32),
    )(p, z)


def _call_pool_head(h, bi, m1, wf1, bf1, m2, wf2, bf2):
    return pl.pallas_call(
        _pool_head,
        out_shape=jax.ShapeDtypeStruct((_G, 1), _f32),
    )(h, bi, m1, wf1, bf1, m2, wf2, bf2)


def kernel(x, edge_index, batch_idx, W1r, b1, W1s, W2r, b2, W2s, W3r, b3, W3s,
           Wf1, bf1, Wf2, bf2, mask1, mask2):
    xp = jnp.pad(x, ((0, _NP - _N), (0, 0)))
    pad = jnp.full((_EP - _E,), _N, jnp.int32)
    src2 = jnp.concatenate([edge_index[0], pad]).reshape(_EP // _CW, _CW)
    dst2 = jnp.concatenate([edge_index[1], pad]).reshape(_EP // _CW, _CW)
    bi = jnp.pad(batch_idx, (0, _NP - _N), constant_values=-1).reshape(_NP, 1)

    def padw(w):
        return jnp.pad(w, ((0, _HP - w.shape[0] if w.shape[0] == _H else 0),
                           (0, _HP - w.shape[1])))
    w1r, w1s = padw(W1r), padw(W1s)
    w2r, w2s = padw(W2r), padw(W2s)
    w3r, w3s = padw(W3r), padw(W3s)
    b1p = jnp.pad(b1, (0, _HP - _H)).reshape(1, _HP)
    b2p = jnp.pad(b2, (0, _HP - _H)).reshape(1, _HP)
    b3p = jnp.pad(b3, (0, _HP - _H)).reshape(1, _HP)
    m1p = jnp.pad(mask1, ((0, 0), (0, _HP - _H)))
    wf1p = jnp.pad(Wf1, ((0, _HP - _H), (0, 0)))
    bf1p = bf1.reshape(1, 30)
    bf2p = bf2.reshape(1, 1)

    Wr = jnp.stack([w1r, w2r, w3r])
    Ws = jnp.stack([w1s, w2s, w3s])
    Bs = jnp.stack([b1p, b2p, b3p])

    def layer(l, h):
        y, z = _call_proj(h, Wr[l], Ws[l], Bs[l])
        p = _seg_sum(y, src2, dst2)
        return _call_combine(p, z)

    h = lax.fori_loop(0, 3, layer, xp)
    return _call_pool_head(h, bi, m1p, wf1p, bf1p, mask2, Wf2, bf2p)


# speedup vs baseline: 5.7509x; 1.4153x over previous
import functools

import jax
import jax.numpy as jnp
from jax import lax
from jax.experimental import pallas as pl
from jax.experimental.pallas import tpu as pltpu
from jax.experimental.pallas import tpu_sc as plsc

_N = 10000
_E = 320000
_F = 128
_H = 66
_G = 64
_HP = 80
_CNT = 66
_NC, _NS = 2, 16
_NW = _NC * _NS
_NP = 10240
_CW = 128
_EP = 327680
_RPW = _EP // _CW // _NW
_RT = _NP // _NS
_K0 = 120

_f32 = jnp.float32


def _seg_body(y, src2, dst2, out, idx_s, idx_d, rows0, rows1, accum,
              isem, sem0, sem1):
    c = lax.axis_index("c")
    s = lax.axis_index("s")

    def _zb(i, carry):
        rows0[i // (_HP // 16), pl.ds((i % (_HP // 16)) * 16, 16)] = (
            jnp.zeros((16,), _f32))
        return carry
    lax.fori_loop(0, _CW * (_HP // 16), _zb, 0)
    base = s * _RT
    for t in range(_RT // _CW):
        pltpu.sync_copy(rows0, accum.at[pl.ds(base + t * _CW, _CW)])
    plsc.subcore_barrier()

    wbase = s * (2 * _RPW) + c * _K0
    nchunks = _K0 - (2 * _K0 - 2 * _RPW) * c

    def _grp(g, carry):
        r0 = wbase + g * 8
        pltpu.async_copy(src2.at[pl.ds(r0, 8)], idx_s, isem)
        pltpu.async_copy(dst2.at[pl.ds(r0, 8)], idx_d, isem)
        pltpu.make_async_copy(src2.at[pl.ds(r0, 8)], idx_s, isem).wait()
        pltpu.make_async_copy(dst2.at[pl.ds(r0, 8)], idx_d, isem).wait()
        pltpu.async_copy(y.at[idx_s.at[0]], rows0, sem0)
        for j in range(8):
            b_rows, b_sem = (rows0, sem0) if j % 2 == 0 else (rows1, sem1)
            n_rows, n_sem = (rows1, sem1) if j % 2 == 0 else (rows0, sem0)
            if j < 7:
                pltpu.async_copy(y.at[idx_s.at[j + 1]], n_rows, n_sem)
            pltpu.make_async_copy(y.at[idx_s.at[j]], b_rows, b_sem).wait()
            pltpu.sync_copy(b_rows, accum.at[idx_d.at[j]], add=True)
        return carry
    lax.fori_loop(0, nchunks // 8, _grp, 0)
    plsc.subcore_barrier()

    for t in range(_RT // _CW):
        r0 = s * _RT + t * _CW
        pltpu.sync_copy(accum.at[pl.ds(r0, _CW)], out.at[c, pl.ds(r0, _CW)])


_seg_sum = functools.partial(
    pl.kernel,
    out_type=jax.ShapeDtypeStruct((_NC, _NP, _HP), _f32),
    mesh=plsc.VectorSubcoreMesh(core_axis_name="c", subcore_axis_name="s",
                                num_cores=_NC, num_subcores=_NS),
    compiler_params=pltpu.CompilerParams(use_tc_tiling_on_sc=False),
    scratch_types=[
        pltpu.VMEM((8, _CW), jnp.int32),
        pltpu.VMEM((8, _CW), jnp.int32),
        pltpu.VMEM((_CW, _HP), _f32),
        pltpu.VMEM((_CW, _HP), _f32),
        pltpu.VMEM_SHARED((_NP, _HP), _f32),
        pltpu.SemaphoreType.DMA,
        pltpu.SemaphoreType.DMA,
        pltpu.SemaphoreType.DMA,
    ],
)(_seg_body)


def _fused(p_ref, z_ref, x_ref, sel_ref, wr_ref, ws_ref, b_ref, y_ref, zo_ref):
    sseg = p_ref[0] + p_ref[1]
    e66c = (lax.broadcasted_iota(jnp.int32, (_HP, 1), 0) == _CNT).astype(_f32)
    cnt = jnp.dot(sseg, e66c, preferred_element_type=_f32)
    inv = 1.0 / jnp.maximum(cnt, 1.0)
    h80 = jnp.maximum(sseg * inv + z_ref[...], 0.0)
    h = jnp.concatenate([h80, jnp.zeros((_NP, _F - _HP), _f32)], axis=1)
    sel = sel_ref[0, 0]
    h = sel * x_ref[...] + (1.0 - sel) * h
    e66r = (lax.broadcasted_iota(jnp.int32, (1, _HP), 1) == _CNT).astype(_f32)
    y_ref[...] = jnp.dot(h, wr_ref[...], preferred_element_type=_f32) + e66r
    zo_ref[...] = jnp.dot(h, ws_ref[...], preferred_element_type=_f32) + b_ref[...]


def _pool_head(p_ref, z_ref, bi_ref, m1_ref, wf1_ref, bf1_ref, m2_ref,
               wf2_ref, bf2_ref, o_ref):
    sseg = p_ref[0] + p_ref[1]
    e66c = (lax.broadcasted_iota(jnp.int32, (_HP, 1), 0) == _CNT).astype(_f32)
    cnt = jnp.dot(sseg, e66c, preferred_element_type=_f32)
    inv = 1.0 / jnp.maximum(cnt, 1.0)
    h = jnp.maximum(sseg * inv + z_ref[...], 0.0)
    oh = (bi_ref[...] == lax.broadcasted_iota(jnp.int32, (_NP, _G), 1)).astype(_f32)
    dn = (((0,), (0,)), ((), ()))
    pool = lax.dot_general(oh, h, dn, preferred_element_type=_f32)
    cnt_g = lax.dot_general(oh, jnp.ones((_NP, 1), _f32), dn,
                            preferred_element_type=_f32)
    f = pool * (1.0 / jnp.maximum(cnt_g, 1.0))
    f = f * m1_ref[...] * 2.0
    f = jnp.dot(f, wf1_ref[...], preferred_element_type=_f32) + bf1_ref[...]
    f = jnp.where(f > 0, f, 0.01 * f)
    f = f * m2_ref[...] * 2.0
    f = jnp.dot(f, wf2_ref[...], preferred_element_type=_f32) + bf2_ref[...]
    o_ref[...] = jnp.where(f > 0, f, 0.01 * f)


def _call_fused(p, z, xp, sel, wr, ws, b):
    return pl.pallas_call(
        _fused,
        out_shape=[jax.ShapeDtypeStruct((_NP, _HP), _f32),
                   jax.ShapeDtypeStruct((_NP, _HP), _f32)],
    )(p, z, xp, sel, wr, ws, b)


def _call_pool_head(p, z, bi, m1, wf1, bf1, m2, wf2, bf2):
    return pl.pallas_call(
        _pool_head,
        out_shape=jax.ShapeDtypeStruct((_G, 1), _f32),
    )(p, z, bi, m1, wf1, bf1, m2, wf2, bf2)


def kernel(x, edge_index, batch_idx, W1r, b1, W1s, W2r, b2, W2s, W3r, b3, W3s,
           Wf1, bf1, Wf2, bf2, mask1, mask2):
    xp = jnp.pad(x, ((0, _NP - _N), (0, 0)))
    pad = jnp.full((_EP - _E,), _N, jnp.int32)
    src2 = jnp.concatenate([edge_index[0], pad]).reshape(_EP // _CW, _CW)
    dst2 = jnp.concatenate([edge_index[1], pad]).reshape(_EP // _CW, _CW)
    bi = jnp.pad(batch_idx, (0, _NP - _N), constant_values=-1).reshape(_NP, 1)

    def padw(w):
        return jnp.pad(w, ((0, _F - w.shape[0]), (0, _HP - w.shape[1])))
    Wr = jnp.stack([padw(W1r), padw(W2r), padw(W3r)])
    Ws = jnp.stack([padw(W1s), padw(W2s), padw(W3s)])

    def padb(b):
        return jnp.pad(b, (0, _HP - _H)).reshape(1, _HP)
    Bs = jnp.stack([padb(b1), padb(b2), padb(b3)])
    m1p = jnp.pad(mask1, ((0, 0), (0, _HP - _H)))
    wf1p = jnp.pad(Wf1, ((0, _HP - _H), (0, 0)))
    bf1p = bf1.reshape(1, 30)
    bf2p = bf2.reshape(1, 1)

    p0 = jnp.zeros((_NC, _NP, _HP), _f32)
    z0 = jnp.zeros((_NP, _HP), _f32)

    def layer(l, carry):
        p, z = carry
        sel = (l == 0).astype(_f32).reshape(1, 1)
        y, z2 = _call_fused(p, z, xp, sel, Wr[l], Ws[l], Bs[l])
        p2 = _seg_sum(y, src2, dst2)
        return (p2, z2)

    p, z = lax.fori_loop(0, 3, layer, (p0, z0))
    return _call_pool_head(p, z, bi, m1p, wf1p, bf1p, mask2, Wf2, bf2p)


# speedup vs baseline: 11.8009x; 2.0520x over previous
import functools

import jax
import jax.numpy as jnp
from jax import lax
from jax.experimental import pallas as pl
from jax.experimental.pallas import tpu as pltpu
from jax.experimental.pallas import tpu_sc as plsc

_N = 10000
_E = 320000
_F = 128
_H = 66
_G = 64
_HP = 80
_CNT = 66
_NC, _NS = 2, 16
_NW = _NC * _NS
_NP = 10240
_CW = 128
_EP = 327680
_RPW = _EP // _CW // _NW
_RT = _NP // _NS
_K0 = 120

_f32 = jnp.float32


def _seg_body(y, src2, dst2, out, idx_s, idx_d, rows0, rows1, rows2, rows3,
              accum, isem, gsem0, gsem1, gsem2, gsem3):
    c = lax.axis_index("c")
    s = lax.axis_index("s")
    rows = (rows0, rows1, rows2, rows3)
    gsem = (gsem0, gsem1, gsem2, gsem3)

    wbase = s * (2 * _RPW) + c * _K0
    nchunks = _K0 - (2 * _K0 - 2 * _RPW) * c
    _K1 = 2 * _RPW - _K0

    @pl.when(c == 0)
    def _():
        pltpu.async_copy(src2.at[pl.ds(wbase, _K0)], idx_s.at[pl.ds(0, _K0)], isem)
        pltpu.async_copy(dst2.at[pl.ds(wbase, _K0)], idx_d.at[pl.ds(0, _K0)], isem)

    @pl.when(c == 1)
    def _():
        pltpu.async_copy(src2.at[pl.ds(wbase, _K1)], idx_s.at[pl.ds(0, _K1)], isem)
        pltpu.async_copy(dst2.at[pl.ds(wbase, _K1)], idx_d.at[pl.ds(0, _K1)], isem)

    def _zb(i, carry):
        rows0[i // (_HP // 16), pl.ds((i % (_HP // 16)) * 16, 16)] = (
            jnp.zeros((16,), _f32))
        return carry
    lax.fori_loop(0, _CW * (_HP // 16), _zb, 0)
    base = s * _RT
    for t in range(_RT // _CW):
        pltpu.sync_copy(rows0, accum.at[pl.ds(base + t * _CW, _CW)])
    plsc.subcore_barrier()

    @pl.when(c == 0)
    def _():
        pltpu.make_async_copy(src2.at[pl.ds(wbase, _K0)],
                              idx_s.at[pl.ds(0, _K0)], isem).wait()
        pltpu.make_async_copy(dst2.at[pl.ds(wbase, _K0)],
                              idx_d.at[pl.ds(0, _K0)], isem).wait()

    @pl.when(c == 1)
    def _():
        pltpu.make_async_copy(src2.at[pl.ds(wbase, _K1)],
                              idx_s.at[pl.ds(0, _K1)], isem).wait()
        pltpu.make_async_copy(dst2.at[pl.ds(wbase, _K1)],
                              idx_d.at[pl.ds(0, _K1)], isem).wait()

    for b in range(3):
        pltpu.async_copy(y.at[idx_s.at[b]], rows[b], gsem[b])

    def _quad(q, carry):
        for b in range(4):
            cc = 4 * q + b

            @pl.when(cc + 3 < nchunks)
            def _():
                pltpu.async_copy(y.at[idx_s.at[cc + 3]], rows[(b + 3) % 4],
                                 gsem[(b + 3) % 4])
            pltpu.make_async_copy(y.at[idx_s.at[cc]], rows[b], gsem[b]).wait()
            pltpu.sync_copy(rows[b], accum.at[idx_d.at[cc]], add=True)
        return carry
    lax.fori_loop(0, nchunks // 4, _quad, 0)
    plsc.subcore_barrier()

    for t in range(_RT // _CW):
        r0 = s * _RT + t * _CW
        pltpu.sync_copy(accum.at[pl.ds(r0, _CW)], out.at[c, pl.ds(r0, _CW)])


_seg_sum = functools.partial(
    pl.kernel,
    out_type=jax.ShapeDtypeStruct((_NC, _NP, _HP), _f32),
    mesh=plsc.VectorSubcoreMesh(core_axis_name="c", subcore_axis_name="s",
                                num_cores=_NC, num_subcores=_NS),
    compiler_params=pltpu.CompilerParams(use_tc_tiling_on_sc=False),
    scratch_types=[
        pltpu.VMEM((_K0, _CW), jnp.int32),
        pltpu.VMEM((_K0, _CW), jnp.int32),
        pltpu.VMEM((_CW, _HP), _f32),
        pltpu.VMEM((_CW, _HP), _f32),
        pltpu.VMEM((_CW, _HP), _f32),
        pltpu.VMEM((_CW, _HP), _f32),
        pltpu.VMEM_SHARED((_NP, _HP), _f32),
        pltpu.SemaphoreType.DMA,
        pltpu.SemaphoreType.DMA,
        pltpu.SemaphoreType.DMA,
        pltpu.SemaphoreType.DMA,
        pltpu.SemaphoreType.DMA,
    ],
)(_seg_body)


def _fused(p_ref, z_ref, x_ref, sel_ref, wr_ref, ws_ref, b_ref, y_ref, zo_ref):
    sseg = p_ref[0] + p_ref[1]
    e66c = (lax.broadcasted_iota(jnp.int32, (_HP, 1), 0) == _CNT).astype(_f32)
    cnt = jnp.dot(sseg, e66c, preferred_element_type=_f32)
    inv = 1.0 / jnp.maximum(cnt, 1.0)
    h80 = jnp.maximum(sseg * inv + z_ref[...], 0.0)
    h = jnp.concatenate([h80, jnp.zeros((_NP, _F - _HP), _f32)], axis=1)
    sel = sel_ref[0, 0]
    h = sel * x_ref[...] + (1.0 - sel) * h
    e66r = (lax.broadcasted_iota(jnp.int32, (1, _HP), 1) == _CNT).astype(_f32)
    y_ref[...] = jnp.dot(h, wr_ref[...], preferred_element_type=_f32) + e66r
    zo_ref[...] = jnp.dot(h, ws_ref[...], preferred_element_type=_f32) + b_ref[...]


def _pool_head(p_ref, z_ref, bi_ref, m1_ref, wf1_ref, bf1_ref, m2_ref,
               wf2_ref, bf2_ref, o_ref):
    sseg = p_ref[0] + p_ref[1]
    e66c = (lax.broadcasted_iota(jnp.int32, (_HP, 1), 0) == _CNT).astype(_f32)
    cnt = jnp.dot(sseg, e66c, preferred_element_type=_f32)
    inv = 1.0 / jnp.maximum(cnt, 1.0)
    h = jnp.maximum(sseg * inv + z_ref[...], 0.0)
    oh = (bi_ref[...] == lax.broadcasted_iota(jnp.int32, (_NP, _G), 1)).astype(_f32)
    dn = (((0,), (0,)), ((), ()))
    pool = lax.dot_general(oh, h, dn, preferred_element_type=_f32)
    cnt_g = lax.dot_general(oh, jnp.ones((_NP, 1), _f32), dn,
                            preferred_element_type=_f32)
    f = pool * (1.0 / jnp.maximum(cnt_g, 1.0))
    f = f * m1_ref[...] * 2.0
    f = jnp.dot(f, wf1_ref[...], preferred_element_type=_f32) + bf1_ref[...]
    f = jnp.where(f > 0, f, 0.01 * f)
    f = f * m2_ref[...] * 2.0
    f = jnp.dot(f, wf2_ref[...], preferred_element_type=_f32) + bf2_ref[...]
    o_ref[...] = jnp.where(f > 0, f, 0.01 * f)


def _call_fused(p, z, xp, sel, wr, ws, b):
    return pl.pallas_call(
        _fused,
        out_shape=[jax.ShapeDtypeStruct((_NP, _HP), _f32),
                   jax.ShapeDtypeStruct((_NP, _HP), _f32)],
    )(p, z, xp, sel, wr, ws, b)


def _call_pool_head(p, z, bi, m1, wf1, bf1, m2, wf2, bf2):
    return pl.pallas_call(
        _pool_head,
        out_shape=jax.ShapeDtypeStruct((_G, 1), _f32),
    )(p, z, bi, m1, wf1, bf1, m2, wf2, bf2)


def kernel(x, edge_index, batch_idx, W1r, b1, W1s, W2r, b2, W2s, W3r, b3, W3s,
           Wf1, bf1, Wf2, bf2, mask1, mask2):
    xp = jnp.pad(x, ((0, _NP - _N), (0, 0)))
    pad = _N + (jnp.arange(_EP - _E, dtype=jnp.int32) % (_NP - _N))
    src2 = jnp.concatenate([edge_index[0], pad]).reshape(_EP // _CW, _CW)
    dst2 = jnp.concatenate([edge_index[1], pad]).reshape(_EP // _CW, _CW)
    bi = jnp.pad(batch_idx, (0, _NP - _N), constant_values=-1).reshape(_NP, 1)

    def padw(w):
        return jnp.pad(w, ((0, _F - w.shape[0]), (0, _HP - w.shape[1])))
    Wr = jnp.stack([padw(W1r), padw(W2r), padw(W3r)])
    Ws = jnp.stack([padw(W1s), padw(W2s), padw(W3s)])

    def padb(b):
        return jnp.pad(b, (0, _HP - _H)).reshape(1, _HP)
    Bs = jnp.stack([padb(b1), padb(b2), padb(b3)])
    m1p = jnp.pad(mask1, ((0, 0), (0, _HP - _H)))
    wf1p = jnp.pad(Wf1, ((0, _HP - _H), (0, 0)))
    bf1p = bf1.reshape(1, 30)
    bf2p = bf2.reshape(1, 1)

    p0 = jnp.zeros((_NC, _NP, _HP), _f32)
    z0 = jnp.zeros((_NP, _HP), _f32)

    def layer(l, carry):
        p, z = carry
        sel = (l == 0).astype(_f32).reshape(1, 1)
        y, z2 = _call_fused(p, z, xp, sel, Wr[l], Ws[l], Bs[l])
        p2 = _seg_sum(y, src2, dst2)
        return (p2, z2)

    p, z = lax.fori_loop(0, 3, layer, (p0, z0))
    return _call_pool_head(p, z, bi, m1p, wf1p, bf1p, mask2, Wf2, bf2p)


# speedup vs baseline: 13.6405x; 1.1559x over previous
import functools

import jax
import jax.numpy as jnp
from jax import lax
from jax.experimental import pallas as pl
from jax.experimental.pallas import tpu as pltpu
from jax.experimental.pallas import tpu_sc as plsc

_N = 10000
_E = 320000
_F = 128
_H = 66
_G = 64
_HP = 80
_CNT = 66
_NC, _NS = 2, 16
_NW = _NC * _NS
_NP = 10240
_CW = 128
_EP = 327680
_RPW = _EP // _CW // _NW
_RT = _NP // _NS
_K0 = 88

_f32 = jnp.float32


def _seg_body(y, src2, dst2, out, idx_s, idx_d, rows0, rows1, rows2, rows3,
              accum, isem, gsem0, gsem1, gsem2, gsem3):
    c = lax.axis_index("c")
    s = lax.axis_index("s")
    rows = (rows0, rows1, rows2, rows3)
    gsem = (gsem0, gsem1, gsem2, gsem3)

    wbase = s * (2 * _RPW) + c * _K0
    nchunks = _K0 - (2 * _K0 - 2 * _RPW) * c
    _K1 = 2 * _RPW - _K0

    @pl.when(c == 0)
    def _():
        pltpu.async_copy(src2.at[pl.ds(wbase, _K0)], idx_s.at[pl.ds(0, _K0)], isem)
        pltpu.async_copy(dst2.at[pl.ds(wbase, _K0)], idx_d.at[pl.ds(0, _K0)], isem)

    @pl.when(c == 1)
    def _():
        pltpu.async_copy(src2.at[pl.ds(wbase, _K1)], idx_s.at[pl.ds(0, _K1)], isem)
        pltpu.async_copy(dst2.at[pl.ds(wbase, _K1)], idx_d.at[pl.ds(0, _K1)], isem)

    def _zb(i, carry):
        rows0[i // (_HP // 16), pl.ds((i % (_HP // 16)) * 16, 16)] = (
            jnp.zeros((16,), _f32))
        return carry
    lax.fori_loop(0, _CW * (_HP // 16), _zb, 0)
    base = s * _RT
    for t in range(_RT // _CW):
        pltpu.sync_copy(rows0, accum.at[pl.ds(base + t * _CW, _CW)])
    plsc.subcore_barrier()

    @pl.when(c == 0)
    def _():
        pltpu.make_async_copy(src2.at[pl.ds(wbase, _K0)],
                              idx_s.at[pl.ds(0, _K0)], isem).wait()
        pltpu.make_async_copy(dst2.at[pl.ds(wbase, _K0)],
                              idx_d.at[pl.ds(0, _K0)], isem).wait()

    @pl.when(c == 1)
    def _():
        pltpu.make_async_copy(src2.at[pl.ds(wbase, _K1)],
                              idx_s.at[pl.ds(0, _K1)], isem).wait()
        pltpu.make_async_copy(dst2.at[pl.ds(wbase, _K1)],
                              idx_d.at[pl.ds(0, _K1)], isem).wait()

    for b in range(3):
        pltpu.async_copy(y.at[idx_s.at[b]], rows[b], gsem[b])

    def _quad(q, carry):
        for b in range(4):
            cc = 4 * q + b

            @pl.when(cc + 3 < nchunks)
            def _():
                pltpu.async_copy(y.at[idx_s.at[cc + 3]], rows[(b + 3) % 4],
                                 gsem[(b + 3) % 4])
            pltpu.make_async_copy(y.at[idx_s.at[cc]], rows[b], gsem[b]).wait()
            pltpu.sync_copy(rows[b], accum.at[idx_d.at[cc]], add=True)
        return carry
    lax.fori_loop(0, nchunks // 4, _quad, 0)
    plsc.subcore_barrier()

    r0 = s * _RT
    pltpu.sync_copy(accum.at[pl.ds(r0, _RT)], out.at[c, pl.ds(r0, _RT)])


_seg_sum = functools.partial(
    pl.kernel,
    out_type=jax.ShapeDtypeStruct((_NC, _NP, _HP), _f32),
    mesh=plsc.VectorSubcoreMesh(core_axis_name="c", subcore_axis_name="s",
                                num_cores=_NC, num_subcores=_NS),
    compiler_params=pltpu.CompilerParams(use_tc_tiling_on_sc=False),
    scratch_types=[
        pltpu.VMEM((max(_K0, 2 * _RPW - _K0), _CW), jnp.int32),
        pltpu.VMEM((max(_K0, 2 * _RPW - _K0), _CW), jnp.int32),
        pltpu.VMEM((_CW, _HP), _f32),
        pltpu.VMEM((_CW, _HP), _f32),
        pltpu.VMEM((_CW, _HP), _f32),
        pltpu.VMEM((_CW, _HP), _f32),
        pltpu.VMEM_SHARED((_NP, _HP), _f32),
        pltpu.SemaphoreType.DMA,
        pltpu.SemaphoreType.DMA,
        pltpu.SemaphoreType.DMA,
        pltpu.SemaphoreType.DMA,
        pltpu.SemaphoreType.DMA,
    ],
)(_seg_body)


def _fused(p_ref, z_ref, x_ref, sel_ref, wr_ref, ws_ref, b_ref, y_ref, zo_ref):
    sseg = p_ref[0] + p_ref[1]
    e66c = (lax.broadcasted_iota(jnp.int32, (_HP, 1), 0) == _CNT).astype(_f32)
    cnt = jnp.dot(sseg, e66c, preferred_element_type=_f32)
    inv = 1.0 / jnp.maximum(cnt, 1.0)
    h80 = jnp.maximum(sseg * inv + z_ref[...], 0.0)
    h = jnp.concatenate([h80, jnp.zeros((_NP, _F - _HP), _f32)], axis=1)
    sel = sel_ref[0, 0]
    h = sel * x_ref[...] + (1.0 - sel) * h
    e66r = (lax.broadcasted_iota(jnp.int32, (1, _HP), 1) == _CNT).astype(_f32)
    y_ref[...] = jnp.dot(h, wr_ref[...], preferred_element_type=_f32) + e66r
    zo_ref[...] = jnp.dot(h, ws_ref[...], preferred_element_type=_f32) + b_ref[...]


def _pool_head(p_ref, z_ref, bi_ref, m1_ref, wf1_ref, bf1_ref, m2_ref,
               wf2_ref, bf2_ref, o_ref):
    sseg = p_ref[0] + p_ref[1]
    e66c = (lax.broadcasted_iota(jnp.int32, (_HP, 1), 0) == _CNT).astype(_f32)
    cnt = jnp.dot(sseg, e66c, preferred_element_type=_f32)
    inv = 1.0 / jnp.maximum(cnt, 1.0)
    h = jnp.maximum(sseg * inv + z_ref[...], 0.0)
    oh = (bi_ref[...] == lax.broadcasted_iota(jnp.int32, (_NP, _G), 1)).astype(_f32)
    dn = (((0,), (0,)), ((), ()))
    pool = lax.dot_general(oh, h, dn, preferred_element_type=_f32)
    cnt_g = lax.dot_general(oh, jnp.ones((_NP, 1), _f32), dn,
                            preferred_element_type=_f32)
    f = pool * (1.0 / jnp.maximum(cnt_g, 1.0))
    f = f * m1_ref[...] * 2.0
    f = jnp.dot(f, wf1_ref[...], preferred_element_type=_f32) + bf1_ref[...]
    f = jnp.where(f > 0, f, 0.01 * f)
    f = f * m2_ref[...] * 2.0
    f = jnp.dot(f, wf2_ref[...], preferred_element_type=_f32) + bf2_ref[...]
    o_ref[...] = jnp.where(f > 0, f, 0.01 * f)


def _call_fused(p, z, xp, sel, wr, ws, b):
    return pl.pallas_call(
        _fused,
        out_shape=[jax.ShapeDtypeStruct((_NP, _HP), _f32),
                   jax.ShapeDtypeStruct((_NP, _HP), _f32)],
    )(p, z, xp, sel, wr, ws, b)


def _call_pool_head(p, z, bi, m1, wf1, bf1, m2, wf2, bf2):
    return pl.pallas_call(
        _pool_head,
        out_shape=jax.ShapeDtypeStruct((_G, 1), _f32),
    )(p, z, bi, m1, wf1, bf1, m2, wf2, bf2)


def kernel(x, edge_index, batch_idx, W1r, b1, W1s, W2r, b2, W2s, W3r, b3, W3s,
           Wf1, bf1, Wf2, bf2, mask1, mask2):
    xp = jnp.pad(x, ((0, _NP - _N), (0, 0)))
    pad = _N + (jnp.arange(_EP - _E, dtype=jnp.int32) % (_NP - _N))
    src2 = jnp.concatenate([edge_index[0], pad]).reshape(_EP // _CW, _CW)
    dst2 = jnp.concatenate([edge_index[1], pad]).reshape(_EP // _CW, _CW)
    bi = jnp.pad(batch_idx, (0, _NP - _N), constant_values=-1).reshape(_NP, 1)

    def padw(w):
        return jnp.pad(w, ((0, _F - w.shape[0]), (0, _HP - w.shape[1])))
    Wr = jnp.stack([padw(W1r), padw(W2r), padw(W3r)])
    Ws = jnp.stack([padw(W1s), padw(W2s), padw(W3s)])

    def padb(b):
        return jnp.pad(b, (0, _HP - _H)).reshape(1, _HP)
    Bs = jnp.stack([padb(b1), padb(b2), padb(b3)])
    m1p = jnp.pad(mask1, ((0, 0), (0, _HP - _H)))
    wf1p = jnp.pad(Wf1, ((0, _HP - _H), (0, 0)))
    bf1p = bf1.reshape(1, 30)
    bf2p = bf2.reshape(1, 1)

    p0 = jnp.zeros((_NC, _NP, _HP), _f32)
    z0 = jnp.zeros((_NP, _HP), _f32)

    def layer(l, carry):
        p, z = carry
        sel = (l == 0).astype(_f32).reshape(1, 1)
        y, z2 = _call_fused(p, z, xp, sel, Wr[l], Ws[l], Bs[l])
        p2 = _seg_sum(y, src2, dst2)
        return (p2, z2)

    p, z = lax.fori_loop(0, 3, layer, (p0, z0))
    return _call_pool_head(p, z, bi, m1p, wf1p, bf1p, mask2, Wf2, bf2p)
